# baseline (speedup 1.0000x reference)
import functools

import jax
import jax.numpy as jnp
from jax import lax
from jax.experimental import pallas as pl
from jax.experimental.pallas import tpu as pltpu
from jax.experimental.pallas import tpu_sc as plsc

N = 10000
E = 320000
D = 128
DA = 144

NC = 2
NS = 16
NW = NC * NS
EP = 327680
EPAD = EP - E
EW = EP // NW
NP = 10240
RPS = NP // NS

CH_D, SB_D = 128, 10
CH_A, SB_A = 80, 16


def _fill_2d(ref, rows, width, value, dtype):
    vl = 16 if dtype == jnp.float32 else 32
    def row(i, _):
        def col(j, _):
            ref[i, pl.ds(j * vl, vl)] = jnp.full((vl,), value, dtype)
            return 0
        lax.fori_loop(0, width // vl, col, 0)
        return 0
    lax.fori_loop(0, rows, row, 0)


def _sc_agg_body(w, ch, sb, dtype, y_hbm, src_hbm, dst_hbm, out_hbm, src_v,
                 dst_v, rows_a, rows_b, acc, sem_ga, sem_gb, sem_sa, sem_sb):
    pairs = sb // 2
    nsb = (EW // ch) // sb
    c = lax.axis_index("c")
    s = lax.axis_index("s")
    wid = c * NS + s

    _fill_2d(rows_a, ch, w, 0.0, dtype)
    for k in range(RPS // ch):
        pltpu.sync_copy(rows_a, acc.at[pl.ds(s * RPS + k * ch, ch)])
    plsc.subcore_barrier()

    def gather(buf, sem, j):
        pltpu.async_copy(y_hbm.at[src_v.at[j]], buf, sem)

    def gather_wait(buf, sem, j):
        pltpu.make_async_copy(y_hbm.at[src_v.at[j]], buf, sem).wait()

    def scat(buf, sem, j):
        pltpu.async_copy(buf, acc.at[dst_v.at[j]], sem, add=True)

    def scat_wait(buf, sem, j):
        pltpu.make_async_copy(buf, acc.at[dst_v.at[j]], sem).wait()

    def superchunk(g, _):
        pltpu.sync_copy(src_hbm.at[wid, g], src_v)
        pltpu.sync_copy(dst_hbm.at[wid, g], dst_v)
        gather(rows_a, sem_ga, 0)
        gather(rows_b, sem_gb, 1)

        def pair(p, _):
            j0 = 2 * p
            j1 = j0 + 1
            gather_wait(rows_a, sem_ga, j0)
            scat(rows_a, sem_sa, j0)
            gather_wait(rows_b, sem_gb, j1)
            scat(rows_b, sem_sb, j1)

            @pl.when(p < pairs - 1)
            def _():
                scat_wait(rows_a, sem_sa, j0)
                gather(rows_a, sem_ga, j0 + 2)
                scat_wait(rows_b, sem_sb, j1)
                gather(rows_b, sem_gb, j1 + 2)
            return 0
        lax.fori_loop(0, pairs, pair, 0)
        scat_wait(rows_a, sem_sa, sb - 2)
        scat_wait(rows_b, sem_sb, sb - 1)
        return 0
    lax.fori_loop(0, nsb, superchunk, 0)

    plsc.subcore_barrier()
    pltpu.sync_copy(acc.at[pl.ds(s * RPS, RPS)],
                    out_hbm.at[c, pl.ds(s * RPS, RPS)])


@functools.lru_cache(maxsize=None)
def _make_sc_agg(w, ch, sb, dtype):
    mesh = plsc.VectorSubcoreMesh(core_axis_name="c", subcore_axis_name="s",
                                  num_cores=NC, num_subcores=NS)
    return pl.kernel(
        functools.partial(_sc_agg_body, w, ch, sb, dtype),
        out_type=[jax.ShapeDtypeStruct((NC, NP, w), dtype)],
        mesh=mesh,
        compiler_params=pltpu.CompilerParams(use_tc_tiling_on_sc=False),
        scratch_types=[
            pltpu.VMEM((sb, ch), jnp.int32),
            pltpu.VMEM((sb, ch), jnp.int32),
            pltpu.VMEM((ch, w), dtype),
            pltpu.VMEM((ch, w), dtype),
            pltpu.VMEM_SHARED((NP, w), dtype),
            pltpu.SemaphoreType.DMA,
            pltpu.SemaphoreType.DMA,
            pltpu.SemaphoreType.DMA,
            pltpu.SemaphoreType.DMA,
        ],
    )


def _combine_math(relu, padded, rb, a, cnt, h_ref, wl_ref, b_ref, wr_ref):
    inv = 1.0 / jnp.maximum(cnt, 1.0)
    m = a * inv
    out = (jnp.dot(m, wl_ref[...], preferred_element_type=jnp.float32)
           + b_ref[...]
           + jnp.dot(h_ref[...], wr_ref[...],
                     preferred_element_type=jnp.float32))
    if relu:
        out = jnp.maximum(out, 0.0)
    if padded:
        rid = (jax.lax.broadcasted_iota(jnp.int32, (rb, 1), 0)
               + pl.program_id(0) * rb)
        out = jnp.where(rid < N, out, 0.0)
    return out


def _tc_combine0_body(rb, acc_ref, h_ref, wl_ref, b_ref, wr_ref, out_ref,
                      outb_ref, cnt_out_ref):
    aug = (acc_ref[0].astype(jnp.float32)
           + acc_ref[1].astype(jnp.float32))
    cnt = aug[:, D:D + 1]
    cnt_out_ref[...] = cnt
    out = _combine_math(True, True, rb, aug[:, :D], cnt, h_ref, wl_ref,
                        b_ref, wr_ref)
    out_ref[...] = out
    outb_ref[...] = out.astype(jnp.bfloat16)


def _tc_combine_body(relu, padded, rb, acc_ref, cnt_ref, h_ref, wl_ref,
                     b_ref, wr_ref, out_ref, outb_ref):
    a = (acc_ref[0].astype(jnp.float32)
         + acc_ref[1].astype(jnp.float32))
    out = _combine_math(relu, padded, rb, a, cnt_ref[...], h_ref, wl_ref,
                        b_ref, wr_ref)
    out_ref[...] = out
    if outb_ref is not None:
        outb_ref[...] = out.astype(jnp.bfloat16)


@functools.lru_cache(maxsize=None)
def _make_combine0():
    rb = 1024
    return pl.pallas_call(
        functools.partial(_tc_combine0_body, rb),
        grid=(NP // rb,),
        in_specs=[
            pl.BlockSpec((NC, rb, DA), lambda i: (0, i, 0)),
            pl.BlockSpec((rb, D), lambda i: (i, 0)),
            pl.BlockSpec((D, D), lambda i: (0, 0)),
            pl.BlockSpec((1, D), lambda i: (0, 0)),
            pl.BlockSpec((D, D), lambda i: (0, 0)),
        ],
        out_specs=[
            pl.BlockSpec((rb, D), lambda i: (i, 0)),
            pl.BlockSpec((rb, D), lambda i: (i, 0)),
            pl.BlockSpec((rb, 1), lambda i: (i, 0)),
        ],
        out_shape=[
            jax.ShapeDtypeStruct((NP, D), jnp.float32),
            jax.ShapeDtypeStruct((NP, D), jnp.bfloat16),
            jax.ShapeDtypeStruct((NP, 1), jnp.float32),
        ],
    )


@functools.lru_cache(maxsize=None)
def _make_combine(relu, padded):
    rb = 1024 if padded else 1000
    n_out = NP if padded else N
    out_specs = [pl.BlockSpec((rb, D), lambda i: (i, 0))]
    out_shape = [jax.ShapeDtypeStruct((n_out, D), jnp.float32)]
    if padded:
        out_specs.append(pl.BlockSpec((rb, D), lambda i: (i, 0)))
        out_shape.append(jax.ShapeDtypeStruct((n_out, D), jnp.bfloat16))
        body = functools.partial(_tc_combine_body, relu, padded, rb)
    else:
        def body(*refs):
            _tc_combine_body(relu, padded, rb, *refs, None)
    return pl.pallas_call(
        body,
        grid=(n_out // rb,),
        in_specs=[
            pl.BlockSpec((NC, rb, D), lambda i: (0, i, 0)),
            pl.BlockSpec((rb, 1), lambda i: (i, 0)),
            pl.BlockSpec((rb, D), lambda i: (i, 0)),
            pl.BlockSpec((D, D), lambda i: (0, 0)),
            pl.BlockSpec((1, D), lambda i: (0, 0)),
            pl.BlockSpec((D, D), lambda i: (0, 0)),
        ],
        out_specs=out_specs,
        out_shape=out_shape,
    )


def kernel(x, edge_index, W_l0, b_l0, W_r0, W_l1, b_l1, W_r1, W_l2, b_l2,
           W_r2):
    PW = EPAD // NW
    k = jnp.arange(PW, dtype=jnp.int32)[None, :]
    w = jnp.arange(NW, dtype=jnp.int32)[:, None]
    pad_src = jnp.broadcast_to(N + k % (NP - N), (NW, PW))
    pad_dst = (w * 313 + k * 41) % N
    r_src = edge_index[0].reshape(NW, E // NW)
    r_dst = edge_index[1].reshape(NW, E // NW)
    src_f = jnp.concatenate([r_src, pad_src], axis=1)
    dst_f = jnp.concatenate([r_dst, pad_dst], axis=1)

    def chunked(a, ch, sb):
        return a.reshape(NW, (EW // ch) // sb, sb, ch)

    src_a, dst_a = chunked(src_f, CH_A, SB_A), chunked(dst_f, CH_A, SB_A)
    src_d, dst_d = chunked(src_f, CH_D, SB_D), chunked(dst_f, CH_D, SB_D)

    ones_col = jnp.concatenate([jnp.ones((N, 1), jnp.float32),
                                jnp.zeros((NP - N, 1), jnp.float32)])
    x_p = jnp.concatenate([x, jnp.zeros((NP - N, D), jnp.float32)])
    x_aug = jnp.concatenate(
        [x_p, ones_col, jnp.zeros((NP, DA - D - 1), jnp.float32)], axis=1)

    agg_a = _make_sc_agg(DA, CH_A, SB_A, jnp.float32)
    agg_d = _make_sc_agg(D, CH_D, SB_D, jnp.bfloat16)
    combine0 = _make_combine0()
    combine_mid = _make_combine(True, True)
    combine_last = _make_combine(False, False)

    (acc2,) = agg_a(x_aug, src_a, dst_a)
    h1, h1b, cnt = combine0(acc2, x_p, W_l0, b_l0.reshape(1, D), W_r0)
    (acc2,) = agg_d(h1b, src_d, dst_d)
    h2, h2b = combine_mid(acc2, cnt, h1, W_l1, b_l1.reshape(1, D), W_r1)
    (acc2,) = agg_d(h2b, src_d, dst_d)
    (out,) = combine_last(acc2, cnt, h2, W_l2, b_l2.reshape(1, D), W_r2)
    return out


# speedup vs baseline: 1.0846x; 1.0846x over previous
import functools

import jax
import jax.numpy as jnp
from jax import lax
from jax.experimental import pallas as pl
from jax.experimental.pallas import tpu as pltpu
from jax.experimental.pallas import tpu_sc as plsc

N = 10000
E = 320000
D = 128
DA = 160

NC = 2
NS = 16
NW = NC * NS
EP = 327680
EPAD = EP - E
EW = EP // NW
NP = 10240
RPS = NP // NS

CH_D, SB_D = 128, 10
CH_A, SB_A = 128, 10


def _fill_2d(ref, rows, width, value):
    def row(i, _):
        def col(j, _):
            ref[i, pl.ds(j * 32, 32)] = jnp.full((32,), value, jnp.bfloat16)
            return 0
        lax.fori_loop(0, width // 32, col, 0)
        return 0
    lax.fori_loop(0, rows, row, 0)


def _sc_agg_body(w, ch, sb, y_hbm, src_hbm, dst_hbm, out_hbm, src_v, dst_v,
                 rows_a, rows_b, acc, sem_ga, sem_gb, sem_sa, sem_sb):
    pairs = sb // 2
    nsb = (EW // ch) // sb
    c = lax.axis_index("c")
    s = lax.axis_index("s")
    wid = c * NS + s

    _fill_2d(rows_a, ch, w, 0.0)
    for k in range(RPS // ch):
        pltpu.sync_copy(rows_a, acc.at[pl.ds(s * RPS + k * ch, ch)])
    plsc.subcore_barrier()

    def gather(buf, sem, j):
        pltpu.async_copy(y_hbm.at[src_v.at[j]], buf, sem)

    def gather_wait(buf, sem, j):
        pltpu.make_async_copy(y_hbm.at[src_v.at[j]], buf, sem).wait()

    def scat(buf, sem, j):
        pltpu.async_copy(buf, acc.at[dst_v.at[j]], sem, add=True)

    def scat_wait(buf, sem, j):
        pltpu.make_async_copy(buf, acc.at[dst_v.at[j]], sem).wait()

    def superchunk(g, _):
        pltpu.sync_copy(src_hbm.at[wid, g], src_v)
        pltpu.sync_copy(dst_hbm.at[wid, g], dst_v)
        gather(rows_a, sem_ga, 0)
        gather(rows_b, sem_gb, 1)

        def pair(p, _):
            j0 = 2 * p
            j1 = j0 + 1
            gather_wait(rows_a, sem_ga, j0)
            scat(rows_a, sem_sa, j0)
            gather_wait(rows_b, sem_gb, j1)
            scat(rows_b, sem_sb, j1)

            @pl.when(p < pairs - 1)
            def _():
                scat_wait(rows_a, sem_sa, j0)
                gather(rows_a, sem_ga, j0 + 2)
                scat_wait(rows_b, sem_sb, j1)
                gather(rows_b, sem_gb, j1 + 2)
            return 0
        lax.fori_loop(0, pairs, pair, 0)
        scat_wait(rows_a, sem_sa, sb - 2)
        scat_wait(rows_b, sem_sb, sb - 1)
        return 0
    lax.fori_loop(0, nsb, superchunk, 0)

    plsc.subcore_barrier()
    pltpu.sync_copy(acc.at[pl.ds(s * RPS, RPS)],
                    out_hbm.at[c, pl.ds(s * RPS, RPS)])


@functools.lru_cache(maxsize=None)
def _make_sc_agg(w, ch, sb):
    mesh = plsc.VectorSubcoreMesh(core_axis_name="c", subcore_axis_name="s",
                                  num_cores=NC, num_subcores=NS)
    return pl.kernel(
        functools.partial(_sc_agg_body, w, ch, sb),
        out_type=[jax.ShapeDtypeStruct((NC, NP, w), jnp.bfloat16)],
        mesh=mesh,
        compiler_params=pltpu.CompilerParams(use_tc_tiling_on_sc=False),
        scratch_types=[
            pltpu.VMEM((sb, ch), jnp.int32),
            pltpu.VMEM((sb, ch), jnp.int32),
            pltpu.VMEM((ch, w), jnp.bfloat16),
            pltpu.VMEM((ch, w), jnp.bfloat16),
            pltpu.VMEM_SHARED((NP, w), jnp.bfloat16),
            pltpu.SemaphoreType.DMA,
            pltpu.SemaphoreType.DMA,
            pltpu.SemaphoreType.DMA,
            pltpu.SemaphoreType.DMA,
        ],
    )


def _combine_math(relu, padded, rb, a, cnt, h_ref, wl_ref, b_ref, wr_ref):
    inv = 1.0 / jnp.maximum(cnt, 1.0)
    m = a * inv
    out = (jnp.dot(m, wl_ref[...], preferred_element_type=jnp.float32)
           + b_ref[...]
           + jnp.dot(h_ref[...], wr_ref[...],
                     preferred_element_type=jnp.float32))
    if relu:
        out = jnp.maximum(out, 0.0)
    if padded:
        rid = (jax.lax.broadcasted_iota(jnp.int32, (rb, 1), 0)
               + pl.program_id(0) * rb)
        out = jnp.where(rid < N, out, 0.0)
    return out


def _tc_combine0_body(rb, acc_ref, h_ref, wl_ref, b_ref, wr_ref, out_ref,
                      outb_ref, cnt_out_ref):
    aug = (acc_ref[0].astype(jnp.float32)
           + acc_ref[1].astype(jnp.float32))
    cnt = aug[:, D:D + 1]
    cnt_out_ref[...] = cnt
    out = _combine_math(True, True, rb, aug[:, :D], cnt, h_ref, wl_ref,
                        b_ref, wr_ref)
    out_ref[...] = out
    outb_ref[...] = out.astype(jnp.bfloat16)


def _tc_combine_body(relu, padded, rb, acc_ref, cnt_ref, h_ref, wl_ref,
                     b_ref, wr_ref, out_ref, outb_ref):
    a = (acc_ref[0].astype(jnp.float32)
         + acc_ref[1].astype(jnp.float32))
    out = _combine_math(relu, padded, rb, a, cnt_ref[...], h_ref, wl_ref,
                        b_ref, wr_ref)
    out_ref[...] = out
    if outb_ref is not None:
        outb_ref[...] = out.astype(jnp.bfloat16)


@functools.lru_cache(maxsize=None)
def _make_combine0():
    rb = 1024
    return pl.pallas_call(
        functools.partial(_tc_combine0_body, rb),
        grid=(NP // rb,),
        in_specs=[
            pl.BlockSpec((NC, rb, DA), lambda i: (0, i, 0)),
            pl.BlockSpec((rb, D), lambda i: (i, 0)),
            pl.BlockSpec((D, D), lambda i: (0, 0)),
            pl.BlockSpec((1, D), lambda i: (0, 0)),
            pl.BlockSpec((D, D), lambda i: (0, 0)),
        ],
        out_specs=[
            pl.BlockSpec((rb, D), lambda i: (i, 0)),
            pl.BlockSpec((rb, D), lambda i: (i, 0)),
            pl.BlockSpec((rb, 1), lambda i: (i, 0)),
        ],
        out_shape=[
            jax.ShapeDtypeStruct((NP, D), jnp.float32),
            jax.ShapeDtypeStruct((NP, D), jnp.bfloat16),
            jax.ShapeDtypeStruct((NP, 1), jnp.float32),
        ],
    )


@functools.lru_cache(maxsize=None)
def _make_combine(relu, padded):
    rb = 1024 if padded else 1000
    n_out = NP if padded else N
    out_specs = [pl.BlockSpec((rb, D), lambda i: (i, 0))]
    out_shape = [jax.ShapeDtypeStruct((n_out, D), jnp.float32)]
    if padded:
        out_specs.append(pl.BlockSpec((rb, D), lambda i: (i, 0)))
        out_shape.append(jax.ShapeDtypeStruct((n_out, D), jnp.bfloat16))
        body = functools.partial(_tc_combine_body, relu, padded, rb)
    else:
        def body(*refs):
            _tc_combine_body(relu, padded, rb, *refs, None)
    return pl.pallas_call(
        body,
        grid=(n_out // rb,),
        in_specs=[
            pl.BlockSpec((NC, rb, D), lambda i: (0, i, 0)),
            pl.BlockSpec((rb, 1), lambda i: (i, 0)),
            pl.BlockSpec((rb, D), lambda i: (i, 0)),
            pl.BlockSpec((D, D), lambda i: (0, 0)),
            pl.BlockSpec((1, D), lambda i: (0, 0)),
            pl.BlockSpec((D, D), lambda i: (0, 0)),
        ],
        out_specs=out_specs,
        out_shape=out_shape,
    )


def kernel(x, edge_index, W_l0, b_l0, W_r0, W_l1, b_l1, W_r1, W_l2, b_l2,
           W_r2):
    PW = EPAD // NW
    k = jnp.arange(PW, dtype=jnp.int32)[None, :]
    w = jnp.arange(NW, dtype=jnp.int32)[:, None]
    pad_src = jnp.broadcast_to(N + k % (NP - N), (NW, PW))
    pad_dst = (w * 313 + k * 41) % N
    r_src = edge_index[0].reshape(NW, E // NW)
    r_dst = edge_index[1].reshape(NW, E // NW)
    src_f = jnp.concatenate([r_src, pad_src], axis=1)
    dst_f = jnp.concatenate([r_dst, pad_dst], axis=1)

    def chunked(a, ch, sb):
        return a.reshape(NW, (EW // ch) // sb, sb, ch)

    src_a, dst_a = chunked(src_f, CH_A, SB_A), chunked(dst_f, CH_A, SB_A)
    src_d, dst_d = chunked(src_f, CH_D, SB_D), chunked(dst_f, CH_D, SB_D)

    ones_col = jnp.concatenate([jnp.ones((N, 1), jnp.bfloat16),
                                jnp.zeros((NP - N, 1), jnp.bfloat16)])
    x_p = jnp.concatenate([x, jnp.zeros((NP - N, D), jnp.float32)])
    x_aug = jnp.concatenate(
        [x_p.astype(jnp.bfloat16), ones_col,
         jnp.zeros((NP, DA - D - 1), jnp.bfloat16)], axis=1)

    agg_a = _make_sc_agg(DA, CH_A, SB_A)
    agg_d = _make_sc_agg(D, CH_D, SB_D)
    combine0 = _make_combine0()
    combine_mid = _make_combine(True, True)
    combine_last = _make_combine(False, False)

    (acc2,) = agg_a(x_aug, src_a, dst_a)
    h1, h1b, cnt = combine0(acc2, x_p, W_l0, b_l0.reshape(1, D), W_r0)
    (acc2,) = agg_d(h1b, src_d, dst_d)
    h2, h2b = combine_mid(acc2, cnt, h1, W_l1, b_l1.reshape(1, D), W_r1)
    (acc2,) = agg_d(h2b, src_d, dst_d)
    (out,) = combine_last(acc2, cnt, h2, W_l2, b_l2.reshape(1, D), W_r2)
    return out
